# Initial kernel scaffold; baseline (speedup 1.0000x reference)
import jax
import jax.numpy as jnp
from jax.experimental import pallas as pl


def kernel(x, emb, W1, b1, W2, b2):
    raise NotImplementedError("write your pallas kernel here")


# speedup vs baseline: 10.4832x; 10.4832x over previous
import functools

import jax
import jax.numpy as jnp
from jax import lax
from jax.experimental import pallas as pl
from jax.experimental.pallas import tpu as pltpu
from jax.experimental.pallas import tpu_sc as plsc

_VOCAB = 1000000
_D = 32
_H = 64
_B = 16384
_L = 200

_NC = 2
_NS = 16
_NW = _NC * _NS
_B_PER_W = _B // _NW
_CHUNK = 256
_UNROLL = 8


def _pool_body(x_hbm, emb_hbm, out_hbm, idx_v, rows_v, stage_v, sem):
    wid = lax.axis_index("s") * _NC + lax.axis_index("c")
    base = wid * _B_PER_W
    for c in range(_B_PER_W // _CHUNK):
        cbase = base + c * _CHUNK
        pltpu.sync_copy(x_hbm.at[pl.ds(cbase, _CHUNK), :], idx_v)

        def row_body(r, carry):
            cp1 = pltpu.async_copy(
                emb_hbm.at[idx_v.at[r, pl.ds(0, 128)]],
                rows_v.at[pl.ds(0, 128), :], sem)
            cp2 = pltpu.async_copy(
                emb_hbm.at[idx_v.at[r, pl.ds(128, _L - 128)]],
                rows_v.at[pl.ds(128, _L - 128), :], sem)
            cp1.wait()
            cp2.wait()

            def red_body(j, accs):
                a0, a1 = accs
                for k in range(_UNROLL):
                    row = j * _UNROLL + k
                    a0 = a0 + rows_v[row, pl.ds(0, 16)]
                    a1 = a1 + rows_v[row, pl.ds(16, 16)]
                return a0, a1

            z = jnp.zeros((16,), jnp.float32)
            a0, a1 = lax.fori_loop(0, _L // _UNROLL, red_body, (z, z))
            stage_v[r, pl.ds(0, 16)] = a0
            stage_v[r, pl.ds(16, 16)] = a1
            return carry

        lax.fori_loop(0, _CHUNK, row_body, 0)
        pltpu.sync_copy(stage_v, out_hbm.at[pl.ds(cbase, _CHUNK), :])


_pool = functools.partial(
    pl.kernel,
    mesh=plsc.VectorSubcoreMesh(core_axis_name="c", subcore_axis_name="s"),
    out_type=jax.ShapeDtypeStruct((_B, _D), jnp.float32),
    scratch_types=[
        pltpu.VMEM((_CHUNK, _L), jnp.int32),
        pltpu.VMEM((_L, _D), jnp.float32),
        pltpu.VMEM((_CHUNK, _D), jnp.float32),
        pltpu.SemaphoreType.DMA,
    ],
    compiler_params=pltpu.CompilerParams(use_tc_tiling_on_sc=False),
)(_pool_body)


def _mlp_body(s_ref, w1_ref, b1_ref, w2_ref, b2_ref, o_ref):
    s = s_ref[...] * (1.0 / _L)
    h = jnp.dot(s, w1_ref[...], preferred_element_type=jnp.float32)
    h = jnp.maximum(h + b1_ref[...], 0.0)
    o = jnp.dot(h, w2_ref[...], preferred_element_type=jnp.float32)
    o_ref[...] = jax.nn.sigmoid(o + b2_ref[...])


def _mlp(pooled, w1, b1, w2, b2):
    return pl.pallas_call(
        _mlp_body,
        out_shape=jax.ShapeDtypeStruct((_B, 1), jnp.float32),
    )(pooled, w1, b1, w2, b2)


def kernel(x, emb, W1, b1, W2, b2):
    pooled = _pool(x, emb)
    return _mlp(pooled, W1, b1.reshape(1, _H), W2, b2.reshape(1, 1))


# speedup vs baseline: 13.8594x; 1.3221x over previous
import functools

import jax
import jax.numpy as jnp
from jax import lax
from jax.experimental import pallas as pl
from jax.experimental.pallas import tpu as pltpu
from jax.experimental.pallas import tpu_sc as plsc

_VOCAB = 1000000
_D = 32
_H = 64
_B = 16384
_L = 200

_NC = 2
_NS = 16
_NW = _NC * _NS
_B_PER_W = _B // _NW
_CHUNK = 256
_UNROLL = 25


def _pool_body(x_hbm, emb_hbm, out_hbm, idx_v, rows0_v, rows1_v, stage_v,
               sem0, sem1):
    wid = lax.axis_index("s") * _NC + lax.axis_index("c")
    base = wid * _B_PER_W
    rows = (rows0_v, rows1_v)
    sems = (sem0, sem1)

    def gather(r, b):
        c1 = pltpu.make_async_copy(
            emb_hbm.at[idx_v.at[r, pl.ds(0, 128)]],
            rows[b].at[pl.ds(0, 128), :], sems[b])
        c2 = pltpu.make_async_copy(
            emb_hbm.at[idx_v.at[r, pl.ds(128, _L - 128)]],
            rows[b].at[pl.ds(128, _L - 128), :], sems[b])
        return c1, c2

    def reduce_store(r, b):
        def red_body(j, accs):
            a0, a1, a2, a3 = accs
            for k in range(_UNROLL):
                row = j * _UNROLL + k
                if k % 2 == 0:
                    a0 = a0 + rows[b][row, pl.ds(0, 16)]
                    a1 = a1 + rows[b][row, pl.ds(16, 16)]
                else:
                    a2 = a2 + rows[b][row, pl.ds(0, 16)]
                    a3 = a3 + rows[b][row, pl.ds(16, 16)]
            return a0, a1, a2, a3

        z = jnp.zeros((16,), jnp.float32)
        a0, a1, a2, a3 = lax.fori_loop(0, _L // _UNROLL, red_body,
                                       (z, z, z, z))
        stage_v[r, pl.ds(0, 16)] = a0 + a2
        stage_v[r, pl.ds(16, 16)] = a1 + a3

    for c in range(_B_PER_W // _CHUNK):
        cbase = base + c * _CHUNK
        pltpu.sync_copy(x_hbm.at[pl.ds(cbase, _CHUNK), :], idx_v)
        for b in range(2):
            c1, c2 = gather(b, b)
            c1.start()
            c2.start()

        def pair_body(i, carry):
            for b in range(2):
                r = 2 * i + b
                c1, c2 = gather(r, b)
                c1.wait()
                c2.wait()
                reduce_store(r, b)

                @pl.when(r + 2 < _CHUNK)
                def _():
                    n1, n2 = gather(r + 2, b)
                    n1.start()
                    n2.start()
            return carry

        lax.fori_loop(0, _CHUNK // 2, pair_body, 0)
        pltpu.sync_copy(stage_v, out_hbm.at[pl.ds(cbase, _CHUNK), :])


_pool = functools.partial(
    pl.kernel,
    mesh=plsc.VectorSubcoreMesh(core_axis_name="c", subcore_axis_name="s"),
    out_type=jax.ShapeDtypeStruct((_B, _D), jnp.float32),
    scratch_types=[
        pltpu.VMEM((_CHUNK, _L), jnp.int32),
        pltpu.VMEM((_L, _D), jnp.float32),
        pltpu.VMEM((_L, _D), jnp.float32),
        pltpu.VMEM((_CHUNK, _D), jnp.float32),
        pltpu.SemaphoreType.DMA,
        pltpu.SemaphoreType.DMA,
    ],
    compiler_params=pltpu.CompilerParams(use_tc_tiling_on_sc=False),
)(_pool_body)


def _mlp_body(s_ref, w1_ref, b1_ref, w2_ref, b2_ref, o_ref):
    s = s_ref[...] * (1.0 / _L)
    h = jnp.dot(s, w1_ref[...], preferred_element_type=jnp.float32)
    h = jnp.maximum(h + b1_ref[...], 0.0)
    o = jnp.dot(h, w2_ref[...], preferred_element_type=jnp.float32)
    o_ref[...] = jax.nn.sigmoid(o + b2_ref[...])


def _mlp(pooled, w1, b1, w2, b2):
    return pl.pallas_call(
        _mlp_body,
        out_shape=jax.ShapeDtypeStruct((_B, 1), jnp.float32),
    )(pooled, w1, b1, w2, b2)


def kernel(x, emb, W1, b1, W2, b2):
    pooled = _pool(x, emb)
    return _mlp(pooled, W1, b1.reshape(1, _H), W2, b2.reshape(1, 1))
